# baseline (speedup 1.0000x reference)
import functools

import jax
import jax.numpy as jnp
from jax import lax
from jax.experimental import pallas as pl
from jax.experimental.pallas import tpu as pltpu
from jax.experimental.pallas import tpu_sc as plsc

N = 10000
E = 320000
D_FEAT = 128
EMBED = 64
B = 4096

NC = 2
NS = 16
NW = NC * NS
L = 16

N_PAD = 10240
ROWS_PER_TILE = N_PAD // NS
CH = 125
NCHUNK = 80
ZCH = 128
B_PER_W = B // NW
NGRP = B_PER_W // L

_mesh = plsc.VectorSubcoreMesh(
    core_axis_name="c", subcore_axis_name="s", num_cores=NC, num_subcores=NS
)
_sc_params = pltpu.CompilerParams(use_tc_tiling_on_sc=False, needs_layout_passes=False)

f32 = jnp.float32
i32 = jnp.int32


@functools.partial(
    pl.kernel,
    out_type=jax.ShapeDtypeStruct((NC, N_PAD), f32),
    mesh=_mesh,
    compiler_params=_sc_params,
    scratch_types=[
        pltpu.VMEM((NCHUNK, CH), i32),
        pltpu.VMEM((ZCH,), f32),
        pltpu.VMEM((ROWS_PER_TILE,), f32),
        pltpu.VMEM_SHARED((N_PAD,), f32),
        pltpu.SemaphoreType.DMA,
    ],
)
def _deg_kernel(dst3, degp, dst_v, ones_v, z_v, deg_sh, sem_d):
    c = lax.axis_index("c")
    s = lax.axis_index("s")
    wid = s * NC + c
    one16 = jnp.ones((L,), f32)
    zero16 = jnp.zeros((L,), f32)
    for k in range(ZCH // L):
        ones_v[pl.ds(k * L, L)] = one16
    for k in range(ROWS_PER_TILE // L):
        z_v[pl.ds(k * L, L)] = zero16
    pltpu.sync_copy(z_v, deg_sh.at[pl.ds(s * ROWS_PER_TILE, ROWS_PER_TILE)])
    plsc.subcore_barrier()
    pltpu.sync_copy(dst3.at[wid], dst_v)

    ones_s = ones_v.at[pl.ds(0, CH)]
    DEPTH = 4

    def body(j, _):
        pltpu.async_copy(ones_s, deg_sh.at[dst_v.at[j]], sem_d, add=True)

        @pl.when(j >= DEPTH)
        def _():
            pltpu.make_async_copy(
                ones_s, deg_sh.at[dst_v.at[j - DEPTH]], sem_d
            ).wait()

        return 0

    lax.fori_loop(0, NCHUNK, body, 0)
    for b in range(DEPTH):
        pltpu.make_async_copy(
            ones_s, deg_sh.at[dst_v.at[NCHUNK - DEPTH + b]], sem_d
        ).wait()
    plsc.subcore_barrier()
    sl = pl.ds(s * ROWS_PER_TILE, ROWS_PER_TILE)
    pltpu.sync_copy(deg_sh.at[sl], degp.at[c, sl])


RB = 1024


def _mm_body(feat_ref, w_ref, u_ref):
    u_ref[...] = jnp.dot(feat_ref[...], w_ref[...], preferred_element_type=f32)


def _matmul(features, gcn_w):
    grid = (N_PAD // RB,)
    return pl.pallas_call(
        _mm_body,
        grid=grid,
        in_specs=[
            pl.BlockSpec((RB, D_FEAT), lambda i: (i, 0)),
            pl.BlockSpec((D_FEAT, EMBED), lambda i: (0, 0)),
        ],
        out_specs=pl.BlockSpec((RB, EMBED), lambda i: (i, 0)),
        out_shape=jax.ShapeDtypeStruct((N_PAD, EMBED), f32),
    )(features, gcn_w)


def _scale_body(u_ref, degp_ref, hp_ref, dinv_ref):
    deg = degp_ref[0, :] + degp_ref[1, :] + 1.0
    di = lax.rsqrt(jnp.maximum(deg, 1.0))
    hp_ref[...] = u_ref[...] * di[:, None]
    dinv_ref[...] = di.reshape(N_PAD // 128, 128)


def _scale(u, degp):
    hp, dinv2 = pl.pallas_call(
        _scale_body,
        in_specs=[
            pl.BlockSpec((N_PAD, EMBED), lambda: (0, 0)),
            pl.BlockSpec((NC, N_PAD), lambda: (0, 0)),
        ],
        out_specs=[
            pl.BlockSpec((N_PAD, EMBED), lambda: (0, 0)),
            pl.BlockSpec((N_PAD // 128, 128), lambda: (0, 0)),
        ],
        out_shape=[
            jax.ShapeDtypeStruct((N_PAD, EMBED), f32),
            jax.ShapeDtypeStruct((N_PAD // 128, 128), f32),
        ],
    )(u, degp)
    return hp, dinv2.reshape(N_PAD)


NB = 5


@functools.partial(
    pl.kernel,
    out_type=[
        jax.ShapeDtypeStruct((N_PAD, EMBED), f32),
        jax.ShapeDtypeStruct((N_PAD, EMBED), f32),
    ],
    mesh=_mesh,
    compiler_params=_sc_params,
    scratch_types=[
        pltpu.VMEM((NCHUNK, CH), i32),
        pltpu.VMEM((NCHUNK, CH), i32),
        pltpu.VMEM((CH, EMBED), f32),
        pltpu.VMEM((CH, EMBED), f32),
        pltpu.VMEM((CH, EMBED), f32),
        pltpu.VMEM((CH, EMBED), f32),
        pltpu.VMEM((CH, EMBED), f32),
        pltpu.VMEM((ZCH, EMBED), f32),
        pltpu.VMEM_SHARED((N_PAD, EMBED), f32),
        pltpu.SemaphoreType.DMA,
        pltpu.SemaphoreType.DMA,
        pltpu.SemaphoreType.DMA,
        pltpu.SemaphoreType.DMA,
        pltpu.SemaphoreType.DMA,
        pltpu.SemaphoreType.DMA,
        pltpu.SemaphoreType.DMA,
        pltpu.SemaphoreType.DMA,
        pltpu.SemaphoreType.DMA,
        pltpu.SemaphoreType.DMA,
    ],
)
def _edge_kernel(src3, dst3, hp, s0_out, s1_out, src_v, dst_v,
                 rows0, rows1, rows2, rows3, rows4, zrows, s_sh,
                 sg0, sg1, sg2, sg3, sg4, ss0, ss1, ss2, ss3, ss4):
    c = lax.axis_index("c")
    s = lax.axis_index("s")
    wid = s * NC + c
    rows = (rows0, rows1, rows2, rows3, rows4)
    sg = (sg0, sg1, sg2, sg3, sg4)
    ss = (ss0, ss1, ss2, ss3, ss4)
    zero16 = jnp.zeros((L,), f32)
    for r in range(ZCH):
        for k in range(EMBED // L):
            zrows[r, pl.ds(k * L, L)] = zero16
    pltpu.async_copy(src3.at[wid], src_v, sg0)
    pltpu.async_copy(dst3.at[wid], dst_v, sg1)
    for k in range(ROWS_PER_TILE // ZCH):
        pltpu.sync_copy(
            zrows, s_sh.at[pl.ds(s * ROWS_PER_TILE + k * ZCH, ZCH), :]
        )
    pltpu.make_async_copy(src3.at[wid], src_v, sg0).wait()
    pltpu.make_async_copy(dst3.at[wid], dst_v, sg1).wait()
    plsc.subcore_barrier()

    def gather(j, b):
        pltpu.async_copy(hp.at[src_v.at[j]], rows[b], sg[b])

    def wait_gather(j, b):
        pltpu.make_async_copy(hp.at[src_v.at[j]], rows[b], sg[b]).wait()

    def scat(j, b):
        pltpu.async_copy(rows[b], s_sh.at[dst_v.at[j]], ss[b], add=True)

    def wait_scat(j, b):
        pltpu.make_async_copy(rows[b], s_sh.at[dst_v.at[j]], ss[b]).wait()

    for b in range(NB):
        gather(b, b)

    def body(jj, _):
        for b in range(NB):
            j = NB * jj + b
            wait_gather(j, b)
            scat(j, b)

            @pl.when(jj < NCHUNK // NB - 1)
            def _():
                wait_scat(j, b)
                gather(j + NB, b)

        return 0

    lax.fori_loop(0, NCHUNK // NB, body, 0)
    for b in range(NB):
        wait_scat(NCHUNK - NB + b, b)
    plsc.subcore_barrier()
    sl = pl.ds(s * ROWS_PER_TILE, ROWS_PER_TILE)

    @pl.when(c == 0)
    def _():
        pltpu.sync_copy(s_sh.at[sl], s0_out.at[sl])

    @pl.when(c == 1)
    def _():
        pltpu.sync_copy(s_sh.at[sl], s1_out.at[sl])


@functools.partial(
    pl.kernel,
    out_type=jax.ShapeDtypeStruct((NW, B_PER_W), f32),
    mesh=_mesh,
    compiler_params=_sc_params,
    scratch_types=[
        pltpu.VMEM((B_PER_W,), i32),
        pltpu.VMEM((B_PER_W,), i32),
        pltpu.VMEM((B_PER_W, EMBED), f32),
        pltpu.VMEM((B_PER_W, EMBED), f32),
        pltpu.VMEM((B_PER_W, EMBED), f32),
        pltpu.VMEM((B_PER_W, EMBED), f32),
        pltpu.VMEM((B_PER_W, EMBED), f32),
        pltpu.VMEM((B_PER_W, EMBED), f32),
        pltpu.VMEM((N_PAD,), f32),
        pltpu.VMEM((N_PAD,), f32),
        pltpu.VMEM((EMBED,), f32),
        pltpu.VMEM((B_PER_W,), f32),
        pltpu.SemaphoreType.DMA,
        pltpu.SemaphoreType.DMA,
    ],
)
def _pair_kernel(pi3, pj3, s0, s1, hp, dinv, lwp, gcn_b, outp,
                 iv, jv, r0i, r1i, rhi, r0j, r1j, rhj, dinv_v, lw_v, b_v, out_v,
                 sem_t, sem_r):
    c = lax.axis_index("c")
    s = lax.axis_index("s")
    wid = s * NC + c
    pltpu.async_copy(pi3.at[wid], iv, sem_t)
    pltpu.async_copy(pj3.at[wid], jv, sem_t)
    pltpu.async_copy(dinv, dinv_v, sem_t)
    pltpu.async_copy(lwp, lw_v, sem_t)
    pltpu.async_copy(gcn_b, b_v, sem_t)
    pltpu.make_async_copy(pi3.at[wid], iv, sem_t).wait()
    pltpu.make_async_copy(pj3.at[wid], jv, sem_t).wait()
    pltpu.async_copy(s0.at[iv], r0i, sem_r)
    pltpu.async_copy(s1.at[iv], r1i, sem_r)
    pltpu.async_copy(hp.at[iv], rhi, sem_r)
    pltpu.async_copy(s0.at[jv], r0j, sem_r)
    pltpu.async_copy(s1.at[jv], r1j, sem_r)
    pltpu.async_copy(hp.at[jv], rhj, sem_r)
    pltpu.make_async_copy(dinv, dinv_v, sem_t).wait()
    pltpu.make_async_copy(lwp, lw_v, sem_t).wait()
    pltpu.make_async_copy(gcn_b, b_v, sem_t).wait()
    pltpu.make_async_copy(s0.at[iv], r0i, sem_r).wait()
    pltpu.make_async_copy(s1.at[iv], r1i, sem_r).wait()
    pltpu.make_async_copy(hp.at[iv], rhi, sem_r).wait()
    pltpu.make_async_copy(s0.at[jv], r0j, sem_r).wait()
    pltpu.make_async_copy(s1.at[jv], r1j, sem_r).wait()
    pltpu.make_async_copy(hp.at[jv], rhj, sem_r).wait()

    lane = lax.iota(i32, L)
    for g in range(NGRP):
        idx_i = iv[pl.ds(g * L, L)]
        idx_j = jv[pl.ds(g * L, L)]
        di = plsc.load_gather(dinv_v, [idx_i])
        dj = plsc.load_gather(dinv_v, [idx_j])
        wi = plsc.load_gather(lw_v, [idx_i])
        wj = plsc.load_gather(lw_v, [idx_j])
        row = lane + g * L

        def fstep(f, acc):
            rot = (jnp.full((L,), f, i32) + lane) & (EMBED - 1)
            ci = (plsc.load_gather(r0i, [row, rot])
                  + plsc.load_gather(r1i, [row, rot])
                  + plsc.load_gather(rhi, [row, rot]))
            cj = (plsc.load_gather(r0j, [row, rot])
                  + plsc.load_gather(r1j, [row, rot])
                  + plsc.load_gather(rhj, [row, rot]))
            bf = plsc.load_gather(b_v, [rot])
            ei = di * ci + bf
            ej = dj * cj + bf
            return acc + ei * ej

        acc = lax.fori_loop(0, EMBED, fstep, jnp.zeros((L,), f32))
        out_v[pl.ds(g * L, L)] = wi + wj + acc
    pltpu.sync_copy(out_v, outp.at[wid])


@jax.jit
def kernel(features, edge_index, interaction_pairs, gcn_w, gcn_b,
           linear_w, linear_bias):
    src3 = edge_index[0].astype(i32).reshape(NW, NCHUNK, CH)
    dst3 = edge_index[1].astype(i32).reshape(NW, NCHUNK, CH)

    lwp = jnp.concatenate([linear_w[:, 0], jnp.zeros((N_PAD - N,), f32)])

    pi3 = interaction_pairs[:, 0].astype(i32).reshape(NW, B_PER_W)
    pj3 = interaction_pairs[:, 1].astype(i32).reshape(NW, B_PER_W)

    degp = _deg_kernel(dst3)
    u = _matmul(features, gcn_w)
    hp, dinv = _scale(u, degp)
    s0, s1 = _edge_kernel(src3, dst3, hp)
    outp = _pair_kernel(pi3, pj3, s0, s1, hp, dinv, lwp, gcn_b)
    return outp.reshape(B) + linear_bias[0]


# speedup vs baseline: 1.0295x; 1.0295x over previous
import functools

import jax
import jax.numpy as jnp
from jax import lax
from jax.experimental import pallas as pl
from jax.experimental.pallas import tpu as pltpu
from jax.experimental.pallas import tpu_sc as plsc

N = 10000
E = 320000
D_FEAT = 128
EMBED = 64
B = 4096

NC = 2
NS = 16
NW = NC * NS
L = 16

N_PAD = 10240
ROWS_PER_TILE = N_PAD // NS
CH = 125
NCHUNK = 80
ZCH = 128
B_PER_W = B // NW
NGRP = B_PER_W // L

_mesh = plsc.VectorSubcoreMesh(
    core_axis_name="c", subcore_axis_name="s", num_cores=NC, num_subcores=NS
)
_sc_params = pltpu.CompilerParams(use_tc_tiling_on_sc=False, needs_layout_passes=False)

f32 = jnp.float32
i32 = jnp.int32


@functools.partial(
    pl.kernel,
    out_type=jax.ShapeDtypeStruct((NC, N_PAD), f32),
    mesh=_mesh,
    compiler_params=_sc_params,
    scratch_types=[
        pltpu.VMEM((NCHUNK, CH), i32),
        pltpu.VMEM((ZCH,), f32),
        pltpu.VMEM((ROWS_PER_TILE,), f32),
        pltpu.VMEM_SHARED((N_PAD,), f32),
        pltpu.SemaphoreType.DMA,
    ],
)
def _deg_kernel(dst3, degp, dst_v, ones_v, z_v, deg_sh, sem_d):
    c = lax.axis_index("c")
    s = lax.axis_index("s")
    wid = s * NC + c
    one16 = jnp.ones((L,), f32)
    zero16 = jnp.zeros((L,), f32)
    for k in range(ZCH // L):
        ones_v[pl.ds(k * L, L)] = one16
    for k in range(ROWS_PER_TILE // L):
        z_v[pl.ds(k * L, L)] = zero16
    pltpu.sync_copy(z_v, deg_sh.at[pl.ds(s * ROWS_PER_TILE, ROWS_PER_TILE)])
    plsc.subcore_barrier()
    pltpu.sync_copy(dst3.at[wid], dst_v)

    ones_s = ones_v.at[pl.ds(0, CH)]
    DEPTH = 4

    def body(j, _):
        pltpu.async_copy(ones_s, deg_sh.at[dst_v.at[j]], sem_d, add=True)

        @pl.when(j >= DEPTH)
        def _():
            pltpu.make_async_copy(
                ones_s, deg_sh.at[dst_v.at[j - DEPTH]], sem_d
            ).wait()

        return 0

    lax.fori_loop(0, NCHUNK, body, 0)
    for b in range(DEPTH):
        pltpu.make_async_copy(
            ones_s, deg_sh.at[dst_v.at[NCHUNK - DEPTH + b]], sem_d
        ).wait()
    plsc.subcore_barrier()
    sl = pl.ds(s * ROWS_PER_TILE, ROWS_PER_TILE)
    pltpu.sync_copy(deg_sh.at[sl], degp.at[c, sl])


RB = 1024


def _mmscale_body(feat_ref, w_ref, degp_ref, hp_ref, dinv_ref):
    u = jnp.dot(feat_ref[...], w_ref[...], preferred_element_type=f32)
    deg = degp_ref[0, :] + degp_ref[1, :] + 1.0
    di = lax.rsqrt(jnp.maximum(deg, 1.0))
    hp_ref[...] = u * di[:, None]
    dinv_ref[...] = di.reshape(RB // 128, 128)


def _mmscale(features, gcn_w, degp):
    grid = (N_PAD // RB,)
    hp, dinv2 = pl.pallas_call(
        _mmscale_body,
        grid=grid,
        in_specs=[
            pl.BlockSpec((RB, D_FEAT), lambda i: (i, 0)),
            pl.BlockSpec((D_FEAT, EMBED), lambda i: (0, 0)),
            pl.BlockSpec((NC, RB), lambda i: (0, i)),
        ],
        out_specs=[
            pl.BlockSpec((RB, EMBED), lambda i: (i, 0)),
            pl.BlockSpec((RB // 128, 128), lambda i: (i, 0)),
        ],
        out_shape=[
            jax.ShapeDtypeStruct((N_PAD, EMBED), f32),
            jax.ShapeDtypeStruct((N_PAD // 128, 128), f32),
        ],
    )(features, gcn_w, degp)
    return hp, dinv2.reshape(N_PAD)


NB = 5


@functools.partial(
    pl.kernel,
    out_type=[
        jax.ShapeDtypeStruct((N_PAD, EMBED), f32),
        jax.ShapeDtypeStruct((N_PAD, EMBED), f32),
    ],
    mesh=_mesh,
    compiler_params=_sc_params,
    scratch_types=[
        pltpu.VMEM((NCHUNK, CH), i32),
        pltpu.VMEM((NCHUNK, CH), i32),
        pltpu.VMEM((CH, EMBED), f32),
        pltpu.VMEM((CH, EMBED), f32),
        pltpu.VMEM((CH, EMBED), f32),
        pltpu.VMEM((CH, EMBED), f32),
        pltpu.VMEM((CH, EMBED), f32),
        pltpu.VMEM((ZCH, EMBED), f32),
        pltpu.VMEM_SHARED((N_PAD, EMBED), f32),
        pltpu.SemaphoreType.DMA,
        pltpu.SemaphoreType.DMA,
        pltpu.SemaphoreType.DMA,
        pltpu.SemaphoreType.DMA,
        pltpu.SemaphoreType.DMA,
        pltpu.SemaphoreType.DMA,
        pltpu.SemaphoreType.DMA,
        pltpu.SemaphoreType.DMA,
        pltpu.SemaphoreType.DMA,
        pltpu.SemaphoreType.DMA,
    ],
)
def _edge_kernel(src3, dst3, hp, s0_out, s1_out, src_v, dst_v,
                 rows0, rows1, rows2, rows3, rows4, zrows, s_sh,
                 sg0, sg1, sg2, sg3, sg4, ss0, ss1, ss2, ss3, ss4):
    c = lax.axis_index("c")
    s = lax.axis_index("s")
    wid = s * NC + c
    rows = (rows0, rows1, rows2, rows3, rows4)
    sg = (sg0, sg1, sg2, sg3, sg4)
    ss = (ss0, ss1, ss2, ss3, ss4)
    zero16 = jnp.zeros((L,), f32)
    for r in range(ZCH):
        for k in range(EMBED // L):
            zrows[r, pl.ds(k * L, L)] = zero16
    pltpu.async_copy(src3.at[wid], src_v, sg0)
    pltpu.async_copy(dst3.at[wid], dst_v, sg1)
    for k in range(ROWS_PER_TILE // ZCH):
        pltpu.sync_copy(
            zrows, s_sh.at[pl.ds(s * ROWS_PER_TILE + k * ZCH, ZCH), :]
        )
    pltpu.make_async_copy(src3.at[wid], src_v, sg0).wait()
    pltpu.make_async_copy(dst3.at[wid], dst_v, sg1).wait()
    plsc.subcore_barrier()

    def gather(j, b):
        pltpu.async_copy(hp.at[src_v.at[j]], rows[b], sg[b])

    def wait_gather(j, b):
        pltpu.make_async_copy(hp.at[src_v.at[j]], rows[b], sg[b]).wait()

    def scat(j, b):
        pltpu.async_copy(rows[b], s_sh.at[dst_v.at[j]], ss[b], add=True)

    def wait_scat(j, b):
        pltpu.make_async_copy(rows[b], s_sh.at[dst_v.at[j]], ss[b]).wait()

    for b in range(NB):
        gather(b, b)

    def body(jj, _):
        for b in range(NB):
            j = NB * jj + b
            wait_gather(j, b)
            scat(j, b)

            @pl.when(jj < NCHUNK // NB - 1)
            def _():
                wait_scat(j, b)
                gather(j + NB, b)

        return 0

    lax.fori_loop(0, NCHUNK // NB, body, 0)
    for b in range(NB):
        wait_scat(NCHUNK - NB + b, b)
    plsc.subcore_barrier()
    sl = pl.ds(s * ROWS_PER_TILE, ROWS_PER_TILE)

    @pl.when(c == 0)
    def _():
        pltpu.sync_copy(s_sh.at[sl], s0_out.at[sl])

    @pl.when(c == 1)
    def _():
        pltpu.sync_copy(s_sh.at[sl], s1_out.at[sl])


@functools.partial(
    pl.kernel,
    out_type=jax.ShapeDtypeStruct((NW, B_PER_W), f32),
    mesh=_mesh,
    compiler_params=_sc_params,
    scratch_types=[
        pltpu.VMEM((B_PER_W,), i32),
        pltpu.VMEM((B_PER_W,), i32),
        pltpu.VMEM((B_PER_W, EMBED), f32),
        pltpu.VMEM((B_PER_W, EMBED), f32),
        pltpu.VMEM((B_PER_W, EMBED), f32),
        pltpu.VMEM((B_PER_W, EMBED), f32),
        pltpu.VMEM((B_PER_W, EMBED), f32),
        pltpu.VMEM((B_PER_W, EMBED), f32),
        pltpu.VMEM((N_PAD,), f32),
        pltpu.VMEM((N_PAD,), f32),
        pltpu.VMEM((EMBED,), f32),
        pltpu.VMEM((B_PER_W,), f32),
        pltpu.SemaphoreType.DMA,
        pltpu.SemaphoreType.DMA,
    ],
)
def _pair_kernel(pi3, pj3, s0, s1, hp, dinv, lwp, gcn_b, outp,
                 iv, jv, r0i, r1i, rhi, r0j, r1j, rhj, dinv_v, lw_v, b_v, out_v,
                 sem_t, sem_r):
    c = lax.axis_index("c")
    s = lax.axis_index("s")
    wid = s * NC + c
    pltpu.async_copy(pi3.at[wid], iv, sem_t)
    pltpu.async_copy(pj3.at[wid], jv, sem_t)
    pltpu.async_copy(dinv, dinv_v, sem_t)
    pltpu.async_copy(lwp, lw_v, sem_t)
    pltpu.async_copy(gcn_b, b_v, sem_t)
    pltpu.make_async_copy(pi3.at[wid], iv, sem_t).wait()
    pltpu.make_async_copy(pj3.at[wid], jv, sem_t).wait()
    pltpu.async_copy(s0.at[iv], r0i, sem_r)
    pltpu.async_copy(s1.at[iv], r1i, sem_r)
    pltpu.async_copy(hp.at[iv], rhi, sem_r)
    pltpu.async_copy(s0.at[jv], r0j, sem_r)
    pltpu.async_copy(s1.at[jv], r1j, sem_r)
    pltpu.async_copy(hp.at[jv], rhj, sem_r)
    pltpu.make_async_copy(dinv, dinv_v, sem_t).wait()
    pltpu.make_async_copy(lwp, lw_v, sem_t).wait()
    pltpu.make_async_copy(gcn_b, b_v, sem_t).wait()
    pltpu.make_async_copy(s0.at[iv], r0i, sem_r).wait()
    pltpu.make_async_copy(s1.at[iv], r1i, sem_r).wait()
    pltpu.make_async_copy(hp.at[iv], rhi, sem_r).wait()
    pltpu.make_async_copy(s0.at[jv], r0j, sem_r).wait()
    pltpu.make_async_copy(s1.at[jv], r1j, sem_r).wait()
    pltpu.make_async_copy(hp.at[jv], rhj, sem_r).wait()

    lane = lax.iota(i32, L)
    for g in range(NGRP):
        idx_i = iv[pl.ds(g * L, L)]
        idx_j = jv[pl.ds(g * L, L)]
        di = plsc.load_gather(dinv_v, [idx_i])
        dj = plsc.load_gather(dinv_v, [idx_j])
        wi = plsc.load_gather(lw_v, [idx_i])
        wj = plsc.load_gather(lw_v, [idx_j])
        row = lane + g * L

        def fstep(f, acc):
            rot = (jnp.full((L,), f, i32) + lane) & (EMBED - 1)
            ci = (plsc.load_gather(r0i, [row, rot])
                  + plsc.load_gather(r1i, [row, rot])
                  + plsc.load_gather(rhi, [row, rot]))
            cj = (plsc.load_gather(r0j, [row, rot])
                  + plsc.load_gather(r1j, [row, rot])
                  + plsc.load_gather(rhj, [row, rot]))
            bf = plsc.load_gather(b_v, [rot])
            ei = di * ci + bf
            ej = dj * cj + bf
            return acc + ei * ej

        acc = lax.fori_loop(0, EMBED, fstep, jnp.zeros((L,), f32))
        out_v[pl.ds(g * L, L)] = wi + wj + acc
    pltpu.sync_copy(out_v, outp.at[wid])


@jax.jit
def kernel(features, edge_index, interaction_pairs, gcn_w, gcn_b,
           linear_w, linear_bias):
    src3 = edge_index[0].astype(i32).reshape(NW, NCHUNK, CH)
    dst3 = edge_index[1].astype(i32).reshape(NW, NCHUNK, CH)

    lwp = jnp.concatenate([linear_w[:, 0], jnp.zeros((N_PAD - N,), f32)])

    pi3 = interaction_pairs[:, 0].astype(i32).reshape(NW, B_PER_W)
    pj3 = interaction_pairs[:, 1].astype(i32).reshape(NW, B_PER_W)

    degp = _deg_kernel(dst3)
    hp, dinv = _mmscale(features, gcn_w, degp)
    s0, s1 = _edge_kernel(src3, dst3, hp)
    outp = _pair_kernel(pi3, pj3, s0, s1, hp, dinv, lwp, gcn_b)
    return outp.reshape(B) + linear_bias[0]
